# baseline (speedup 1.0000x reference)
import functools

import jax
import jax.numpy as jnp
from jax import lax
from jax.experimental import pallas as pl
from jax.experimental.pallas import tpu as pltpu
from jax.experimental.pallas import tpu_sc as plsc

N = 10000
E = 160000
H = 256
C = 10
R = 16
G = 64
NR = N * R
EPS = 1e-5

BN_ROWS = 1000
NBLK = N // BN_ROWS
BE = 32000
NEB = E // BE
N_SUB = 16
E_PER_SUB = E // N_SUB
CHUNK = 80
NPAD = 10240
ROWS_PER_SUB = NPAD // N_SUB


def _edge_prep_body(attr_ref, src_ref, g_ref):
    a = jnp.transpose(attr_ref[...])
    mx = jnp.max(a, axis=0, keepdims=True)
    ids = lax.broadcasted_iota(jnp.int32, (R, BE), 0)
    t = jnp.min(jnp.where(a == mx, ids, R), axis=0)
    g = t * N + src_ref[0, 0, :]
    g_ref[0, 0, 0, :] = g
    g_ref[1, 0, 0, :] = g + NR


def _edge_prep(edge_attr, src):
    src3 = src.reshape(NEB, 1, BE)
    g2 = pl.pallas_call(
        _edge_prep_body,
        grid=(NEB,),
        in_specs=[
            pl.BlockSpec((BE, R), lambda i: (i, 0)),
            pl.BlockSpec((1, 1, BE), lambda i: (i, 0, 0)),
        ],
        out_specs=pl.BlockSpec((2, 1, 1, BE), lambda i: (0, i, 0, 0)),
        out_shape=jax.ShapeDtypeStruct((2, NEB, 1, BE), jnp.int32),
    )(edge_attr, src3)
    return g2.reshape(2 * E)


def _bn_relu_full(p, g, bt):
    m = jnp.sum(p, axis=0, keepdims=True) / N
    v = jnp.sum(p * p, axis=0, keepdims=True) / N - m * m
    scale = g * lax.rsqrt(v + EPS)
    return jnp.maximum(p * scale + (bt - m * scale), 0.0)


def _pool_sorted(h, batch_ref):
    pooled = jnp.zeros((G, H), jnp.float32)
    cnts = jnp.zeros((G, 128), jnp.float32)
    ones = jnp.ones((BN_ROWS, 128), jnp.float32)
    for i in range(NBLK):
        b = batch_ref[i, 0, :]
        oh = (lax.broadcasted_iota(jnp.int32, (G, BN_ROWS), 0)
              == b[None, :]).astype(jnp.float32)
        pooled += jnp.dot(oh, h[i * BN_ROWS:(i + 1) * BN_ROWS],
                          preferred_element_type=jnp.float32)
        cnts += jnp.dot(oh, ones, preferred_element_type=jnp.float32)
    return pooled, cnts


def _mlp1_body(x_ref, w1_ref, b1_ref, g1_ref, bt1_ref, w2_ref, b2_ref,
               g2_ref, bt2_ref, batch_ref, h2_ref, pool_ref, cnt_ref):
    p1 = jnp.dot(x_ref[...], w1_ref[...],
                 preferred_element_type=jnp.float32) + b1_ref[...]
    h1 = _bn_relu_full(p1, g1_ref[...], bt1_ref[...])
    p2 = jnp.dot(h1, w2_ref[...],
                 preferred_element_type=jnp.float32) + b2_ref[...]
    h2 = _bn_relu_full(p2, g2_ref[...], bt2_ref[...])
    h2_ref[...] = h2
    pooled, cnts = _pool_sorted(h2, batch_ref)
    pool_ref[...] = pooled
    cnt_ref[...] = cnts


def _mlp1(x, W1, b1, g1, bt1, W2, b2, g2, bt2, batch3):
    return pl.pallas_call(
        _mlp1_body,
        out_shape=[
            jax.ShapeDtypeStruct((N, H), jnp.float32),
            jax.ShapeDtypeStruct((G, H), jnp.float32),
            jax.ShapeDtypeStruct((G, 128), jnp.float32),
        ],
    )(x, W1, b1.reshape(1, -1), g1.reshape(1, -1), bt1.reshape(1, -1),
      W2, b2.reshape(1, -1), g2.reshape(1, -1), bt2.reshape(1, -1), batch3)


def _tail_body(h2_ref, wr_ref, cb_ref, agg_ref, w3_ref, b3_ref, g3_ref,
               bt3_ref, w4_ref, b4_ref, g4_ref, bt4_ref, batch_ref,
               pool2_ref, cnt_ref, w0_ref, bb0_ref, wl1_ref, bb1_ref, o_ref):
    root = jnp.dot(h2_ref[...], wr_ref[...],
                   preferred_element_type=jnp.float32) + cb_ref[...]
    conv = root + jnp.concatenate(
        [agg_ref[0, 0:N, :], agg_ref[1, 0:N, :]], axis=1)
    p3 = jnp.dot(conv, w3_ref[...],
                 preferred_element_type=jnp.float32) + b3_ref[...]
    h3 = _bn_relu_full(p3, g3_ref[...], bt3_ref[...])
    p4 = jnp.dot(h3, w4_ref[...],
                 preferred_element_type=jnp.float32) + b4_ref[...]
    h4 = _bn_relu_full(p4, g4_ref[...], bt4_ref[...])
    pooled4, _ = _pool_sorted(h4, batch_ref)
    out0 = (jnp.dot(pool2_ref[...], w0_ref[...],
                    preferred_element_type=jnp.float32)
            + cnt_ref[:, 0:1] * bb0_ref[...])
    out1 = (jnp.dot(pooled4, wl1_ref[...],
                    preferred_element_type=jnp.float32) + bb1_ref[...])
    logits = out0 + out1
    logits = logits - jnp.max(logits, axis=1, keepdims=True)
    o_ref[...] = logits - jnp.log(jnp.sum(jnp.exp(logits), axis=1,
                                          keepdims=True))


def _tail(h2, Wroot, conv_b, agg, W3, b3, g3, bt3, W4, b4, g4, bt4, batch3,
          pool2, counts, lin0_W, lin0_b, lin1_W, lin1_b):
    return pl.pallas_call(
        _tail_body,
        out_shape=jax.ShapeDtypeStruct((G, C), jnp.float32),
    )(h2, Wroot, conv_b.reshape(1, -1), agg,
      W3, b3.reshape(1, -1), g3.reshape(1, -1), bt3.reshape(1, -1),
      W4, b4.reshape(1, -1), g4.reshape(1, -1), bt4.reshape(1, -1), batch3,
      pool2, counts, lin0_W, lin0_b.reshape(1, -1),
      lin1_W, lin1_b.reshape(1, -1))


def _rel_body(h_ref, wc_ref, t_ref):
    res = jnp.dot(h_ref[...], wc_ref[0], preferred_element_type=jnp.float32)
    for r in range(R):
        t_ref[0, r] = res[:, r * 128:(r + 1) * 128]


def _rel_transform(h2, wcat2):
    return pl.pallas_call(
        _rel_body,
        grid=(2, NBLK),
        in_specs=[
            pl.BlockSpec((BN_ROWS, H), lambda hf, i: (i, 0)),
            pl.BlockSpec((1, H, R * 128), lambda hf, i: (hf, 0, 0)),
        ],
        out_specs=pl.BlockSpec((1, R, BN_ROWS, 128),
                               lambda hf, i: (hf, 0, i, 0)),
        out_shape=jax.ShapeDtypeStruct((2, R, N, 128), jnp.float32),
    )(h2, wcat2)


NCHUNK = E_PER_SUB // CHUNK


def _sc_agg_body(t_hbm, gidx_hbm, dst_hbm, zeros_hbm, out_hbm,
                 gidx_v, rows0, rows1, rows2,
                 dc0, dc1, dc2, acc_sh, sem0, sem1, sem2, dsem0, dsem1, dsem2):
    c = lax.axis_index("c")
    s = lax.axis_index("s")
    z0 = s * ROWS_PER_SUB
    pltpu.sync_copy(zeros_hbm, acc_sh.at[pl.ds(z0, ROWS_PER_SUB)])
    pltpu.sync_copy(gidx_hbm.at[pl.ds(c * E + s * E_PER_SUB, E_PER_SUB)], gidx_v)
    plsc.subcore_barrier()

    rows = (rows0, rows1, rows2)
    dcs = (dc0, dc1, dc2)
    sems = (sem0, sem1, sem2)
    dsems = (dsem0, dsem1, dsem2)

    def gstart(t, p):
        pltpu.async_copy(
            dst_hbm.at[pl.ds(s * E_PER_SUB + t * CHUNK, CHUNK)], dcs[p],
            dsems[p])
        pltpu.async_copy(
            t_hbm.at[gidx_v.at[pl.ds(t * CHUNK, CHUNK)]], rows[p], sems[p])

    def gwait(t, p):
        pltpu.make_async_copy(
            t_hbm.at[gidx_v.at[pl.ds(t * CHUNK, CHUNK)]], rows[p], sems[p]).wait()
        pltpu.make_async_copy(
            dst_hbm.at[pl.ds(s * E_PER_SUB + t * CHUNK, CHUNK)], dcs[p],
            dsems[p]).wait()

    def scatter(p):
        pltpu.sync_copy(rows[p], acc_sh.at[dcs[p]], add=True)

    for q in range(2):
        gstart(q, q)

    @pl.loop(0, NCHUNK - 2, step=3)
    def _(t):
        for q in range(3):
            gwait(t + q, q)
            gstart(t + q + 2, (q + 2) % 3)
            scatter(q)

    gwait(NCHUNK - 2, 0)
    scatter(0)
    gwait(NCHUNK - 1, 1)
    scatter(1)

    plsc.subcore_barrier()
    pltpu.sync_copy(acc_sh.at[pl.ds(z0, ROWS_PER_SUB)],
                    out_hbm.at[pl.ds(c * NPAD + z0, ROWS_PER_SUB)])


def _sc_aggregate(t_flat, gidx2, dst, zeros):
    mesh = plsc.VectorSubcoreMesh(core_axis_name="c", subcore_axis_name="s")
    fn = pl.kernel(
        _sc_agg_body,
        out_type=jax.ShapeDtypeStruct((2 * NPAD, 128), jnp.float32),
        mesh=mesh,
        scratch_types=[
            pltpu.VMEM((E_PER_SUB,), jnp.int32),
            pltpu.VMEM((CHUNK, 128), jnp.float32),
            pltpu.VMEM((CHUNK, 128), jnp.float32),
            pltpu.VMEM((CHUNK, 128), jnp.float32),
            pltpu.VMEM((CHUNK,), jnp.int32),
            pltpu.VMEM((CHUNK,), jnp.int32),
            pltpu.VMEM((CHUNK,), jnp.int32),
            pltpu.VMEM_SHARED((NPAD, 128), jnp.float32),
            pltpu.SemaphoreType.DMA,
            pltpu.SemaphoreType.DMA,
            pltpu.SemaphoreType.DMA,
            pltpu.SemaphoreType.DMA,
            pltpu.SemaphoreType.DMA,
            pltpu.SemaphoreType.DMA,
        ],
    )
    return fn(t_flat, gidx2, dst, zeros)


def kernel(x, edge_index, edge_attr, batch, W1, b1, g1, bt1, W2, b2, g2, bt2,
           lin0_W, lin0_b, Wrel, Wroot, conv_b, W3, b3, g3, bt3,
           W4, b4, g4, bt4, lin1_W, lin1_b):
    src = edge_index[0]
    dst = edge_index[1]
    batch3 = batch.reshape(NBLK, 1, BN_ROWS)

    wcat2 = (Wrel.reshape(R, H, 2, 128).transpose(2, 1, 0, 3)
             .reshape(2, H, R * 128))

    gidx2 = _edge_prep(edge_attr, src)

    h2, pool2, counts = _mlp1(x, W1, b1, g1, bt1, W2, b2, g2, bt2, batch3)

    t_arr = _rel_transform(h2, wcat2)
    t_flat = t_arr.reshape(2 * NR, 128)

    zeros = jnp.zeros((ROWS_PER_SUB, 128), jnp.float32)
    agg = _sc_aggregate(t_flat, gidx2, dst, zeros).reshape(2, NPAD, 128)

    return _tail(h2, Wroot, conv_b, agg, W3, b3, g3, bt3, W4, b4, g4, bt4,
                 batch3, pool2, counts, lin0_W, lin0_b, lin1_W, lin1_b)


# speedup vs baseline: 1.2727x; 1.2727x over previous
import functools

import jax
import jax.numpy as jnp
from jax import lax
from jax.experimental import pallas as pl
from jax.experimental.pallas import tpu as pltpu
from jax.experimental.pallas import tpu_sc as plsc

N = 10000
E = 160000
H = 256
C = 10
R = 16
G = 64
NR = N * R
EPS = 1e-5

BN_ROWS = 1000
NBLK = N // BN_ROWS
BE = 32000
NEB = E // BE
N_SUB = 16
E_PER_SUB = E // N_SUB
CHUNK = 80
NPAD = 10240
ROWS_PER_SUB = NPAD // N_SUB


def _edge_prep_body(attr_ref, src_ref, g_ref):
    a = attr_ref[...]
    mx = jnp.max(a, axis=0, keepdims=True)
    ids = lax.broadcasted_iota(jnp.int32, (R, BE), 0)
    t = jnp.min(jnp.where(a == mx, ids, R), axis=0)
    g = t * N + src_ref[0, 0, :]
    g_ref[0, 0, 0, :] = g
    g_ref[1, 0, 0, :] = g + NR


def _edge_prep(edge_attr, src):
    src3 = src.reshape(NEB, 1, BE)
    attr_t = edge_attr.T
    g2 = pl.pallas_call(
        _edge_prep_body,
        grid=(NEB,),
        in_specs=[
            pl.BlockSpec((R, BE), lambda i: (0, i)),
            pl.BlockSpec((1, 1, BE), lambda i: (i, 0, 0)),
        ],
        out_specs=pl.BlockSpec((2, 1, 1, BE), lambda i: (0, i, 0, 0)),
        out_shape=jax.ShapeDtypeStruct((2, NEB, 1, BE), jnp.int32),
    )(attr_t, src3)
    return g2.reshape(2 * E)


def _bn_relu_full(p, g, bt):
    m = jnp.sum(p, axis=0, keepdims=True) / N
    v = jnp.sum(p * p, axis=0, keepdims=True) / N - m * m
    scale = g * lax.rsqrt(v + EPS)
    return jnp.maximum(p * scale + (bt - m * scale), 0.0)


def _pool_sorted(h, batch_ref):
    pooled = jnp.zeros((G, H), jnp.float32)
    cnts = jnp.zeros((G, 128), jnp.float32)
    ones = jnp.ones((BN_ROWS, 128), jnp.float32)
    for i in range(NBLK):
        b = batch_ref[i, 0, :]
        oh = (lax.broadcasted_iota(jnp.int32, (G, BN_ROWS), 0)
              == b[None, :]).astype(jnp.float32)
        pooled += jnp.dot(oh, h[i * BN_ROWS:(i + 1) * BN_ROWS],
                          preferred_element_type=jnp.float32)
        cnts += jnp.dot(oh, ones, preferred_element_type=jnp.float32)
    return pooled, cnts


def _mlp1_body(x_ref, w1_ref, b1_ref, g1_ref, bt1_ref, w2_ref, b2_ref,
               g2_ref, bt2_ref, batch_ref, h2_ref, pool_ref, cnt_ref):
    p1 = jnp.dot(x_ref[...], w1_ref[...],
                 preferred_element_type=jnp.float32) + b1_ref[...]
    h1 = _bn_relu_full(p1, g1_ref[...], bt1_ref[...])
    p2 = jnp.dot(h1, w2_ref[...],
                 preferred_element_type=jnp.float32) + b2_ref[...]
    h2 = _bn_relu_full(p2, g2_ref[...], bt2_ref[...])
    h2_ref[...] = h2
    pooled, cnts = _pool_sorted(h2, batch_ref)
    pool_ref[...] = pooled
    cnt_ref[...] = cnts


def _mlp1(x, W1, b1, g1, bt1, W2, b2, g2, bt2, batch3):
    return pl.pallas_call(
        _mlp1_body,
        out_shape=[
            jax.ShapeDtypeStruct((N, H), jnp.float32),
            jax.ShapeDtypeStruct((G, H), jnp.float32),
            jax.ShapeDtypeStruct((G, 128), jnp.float32),
        ],
    )(x, W1, b1.reshape(1, -1), g1.reshape(1, -1), bt1.reshape(1, -1),
      W2, b2.reshape(1, -1), g2.reshape(1, -1), bt2.reshape(1, -1), batch3)


def _tail_body(h2_ref, wr_ref, cb_ref, agg_ref, w3_ref, b3_ref, g3_ref,
               bt3_ref, w4_ref, b4_ref, g4_ref, bt4_ref, batch_ref,
               pool2_ref, cnt_ref, w0_ref, bb0_ref, wl1_ref, bb1_ref, o_ref):
    root = jnp.dot(h2_ref[...], wr_ref[...],
                   preferred_element_type=jnp.float32) + cb_ref[...]
    conv = root + jnp.concatenate(
        [agg_ref[0, 0:N, :], agg_ref[1, 0:N, :]], axis=1)
    p3 = jnp.dot(conv, w3_ref[...],
                 preferred_element_type=jnp.float32) + b3_ref[...]
    h3 = _bn_relu_full(p3, g3_ref[...], bt3_ref[...])
    p4 = jnp.dot(h3, w4_ref[...],
                 preferred_element_type=jnp.float32) + b4_ref[...]
    h4 = _bn_relu_full(p4, g4_ref[...], bt4_ref[...])
    pooled4, _ = _pool_sorted(h4, batch_ref)
    out0 = (jnp.dot(pool2_ref[...], w0_ref[...],
                    preferred_element_type=jnp.float32)
            + cnt_ref[:, 0:1] * bb0_ref[...])
    out1 = (jnp.dot(pooled4, wl1_ref[...],
                    preferred_element_type=jnp.float32) + bb1_ref[...])
    logits = out0 + out1
    logits = logits - jnp.max(logits, axis=1, keepdims=True)
    o_ref[...] = logits - jnp.log(jnp.sum(jnp.exp(logits), axis=1,
                                          keepdims=True))


def _tail(h2, Wroot, conv_b, agg, W3, b3, g3, bt3, W4, b4, g4, bt4, batch3,
          pool2, counts, lin0_W, lin0_b, lin1_W, lin1_b):
    return pl.pallas_call(
        _tail_body,
        out_shape=jax.ShapeDtypeStruct((G, C), jnp.float32),
    )(h2, Wroot, conv_b.reshape(1, -1), agg,
      W3, b3.reshape(1, -1), g3.reshape(1, -1), bt3.reshape(1, -1),
      W4, b4.reshape(1, -1), g4.reshape(1, -1), bt4.reshape(1, -1), batch3,
      pool2, counts, lin0_W, lin0_b.reshape(1, -1),
      lin1_W, lin1_b.reshape(1, -1))


def _rel_body(h_ref, wc_ref, t_ref):
    res = jnp.dot(h_ref[...], wc_ref[0], preferred_element_type=jnp.float32)
    for r in range(R):
        t_ref[0, r] = res[:, r * 128:(r + 1) * 128]


def _rel_transform(h2, wcat2):
    return pl.pallas_call(
        _rel_body,
        grid=(2, NBLK),
        in_specs=[
            pl.BlockSpec((BN_ROWS, H), lambda hf, i: (i, 0)),
            pl.BlockSpec((1, H, R * 128), lambda hf, i: (hf, 0, 0)),
        ],
        out_specs=pl.BlockSpec((1, R, BN_ROWS, 128),
                               lambda hf, i: (hf, 0, i, 0)),
        out_shape=jax.ShapeDtypeStruct((2, R, N, 128), jnp.float32),
    )(h2, wcat2)


NCHUNK = E_PER_SUB // CHUNK


def _sc_agg_body(t_hbm, gidx_hbm, dst_hbm, zeros_hbm, out_hbm,
                 rows0, rows1, rows2, rows3, gc0, gc1, gc2, gc3,
                 dc0, dc1, dc2, dc3, acc_sh,
                 sem0, sem1, sem2, sem3, isem0, isem1, isem2, isem3):
    c = lax.axis_index("c")
    s = lax.axis_index("s")
    z0 = s * ROWS_PER_SUB
    pltpu.sync_copy(zeros_hbm, acc_sh.at[pl.ds(z0, ROWS_PER_SUB)])
    plsc.subcore_barrier()

    rows = (rows0, rows1, rows2, rows3)
    gcs = (gc0, gc1, gc2, gc3)
    dcs = (dc0, dc1, dc2, dc3)
    sems = (sem0, sem1, sem2, sem3)
    isems = (isem0, isem1, isem2, isem3)
    gbase = c * E + s * E_PER_SUB
    dbase = s * E_PER_SUB

    def istart(t, p):
        pltpu.async_copy(gidx_hbm.at[pl.ds(gbase + t * CHUNK, CHUNK)],
                         gcs[p], isems[p])
        pltpu.async_copy(dst_hbm.at[pl.ds(dbase + t * CHUNK, CHUNK)],
                         dcs[p], isems[p])

    def iwait(t, p):
        pltpu.make_async_copy(gidx_hbm.at[pl.ds(gbase + t * CHUNK, CHUNK)],
                              gcs[p], isems[p]).wait()
        pltpu.make_async_copy(dst_hbm.at[pl.ds(dbase + t * CHUNK, CHUNK)],
                              dcs[p], isems[p]).wait()

    def gstart(t, p):
        pltpu.async_copy(t_hbm.at[gcs[p]], rows[p], sems[p])

    def gwait(t, p):
        pltpu.make_async_copy(t_hbm.at[gcs[p]], rows[p], sems[p]).wait()

    def scatter(p):
        pltpu.sync_copy(rows[p], acc_sh.at[dcs[p]], add=True)

    for q in range(4):
        istart(q, q)
    for q in range(2):
        iwait(q, q)
        gstart(q, q)

    @pl.loop(0, NCHUNK - 1, step=4)
    def _(t):
        for q in range(4):
            gwait(t + q, q)

            @pl.when(t + q + 2 < NCHUNK)
            def _():
                iwait(t + q + 2, (q + 2) % 4)
                gstart(t + q + 2, (q + 2) % 4)

            scatter(q)

            @pl.when(t + q + 4 < NCHUNK)
            def _():
                istart(t + q + 4, q)

    gwait(NCHUNK - 1, 0)
    scatter(0)

    plsc.subcore_barrier()
    pltpu.sync_copy(acc_sh.at[pl.ds(z0, ROWS_PER_SUB)],
                    out_hbm.at[pl.ds(c * NPAD + z0, ROWS_PER_SUB)])


def _sc_aggregate(t_flat, gidx2, dst, zeros):
    mesh = plsc.VectorSubcoreMesh(core_axis_name="c", subcore_axis_name="s")
    fn = pl.kernel(
        _sc_agg_body,
        out_type=jax.ShapeDtypeStruct((2 * NPAD, 128), jnp.float32),
        mesh=mesh,
        scratch_types=(
            [pltpu.VMEM((CHUNK, 128), jnp.float32)] * 4
            + [pltpu.VMEM((CHUNK,), jnp.int32)] * 8
            + [pltpu.VMEM_SHARED((NPAD, 128), jnp.float32)]
            + [pltpu.SemaphoreType.DMA] * 8
        ),
    )
    return fn(t_flat, gidx2, dst, zeros)


def kernel(x, edge_index, edge_attr, batch, W1, b1, g1, bt1, W2, b2, g2, bt2,
           lin0_W, lin0_b, Wrel, Wroot, conv_b, W3, b3, g3, bt3,
           W4, b4, g4, bt4, lin1_W, lin1_b):
    src = edge_index[0]
    dst = edge_index[1]
    batch3 = batch.reshape(NBLK, 1, BN_ROWS)

    wcat2 = (Wrel.reshape(R, H, 2, 128).transpose(2, 1, 0, 3)
             .reshape(2, H, R * 128))

    gidx2 = _edge_prep(edge_attr, src)

    h2, pool2, counts = _mlp1(x, W1, b1, g1, bt1, W2, b2, g2, bt2, batch3)

    t_arr = _rel_transform(h2, wcat2)
    t_flat = t_arr.reshape(2 * NR, 128)

    zeros = jnp.zeros((ROWS_PER_SUB, 128), jnp.float32)
    agg = _sc_aggregate(t_flat, gidx2, dst, zeros).reshape(2, NPAD, 128)

    return _tail(h2, Wroot, conv_b, agg, W3, b3, g3, bt3, W4, b4, g4, bt4,
                 batch3, pool2, counts, lin0_W, lin0_b, lin1_W, lin1_b)
